# baseline (speedup 1.0000x reference)
import jax
import jax.numpy as jnp
from jax import lax
from jax.experimental import pallas as pl
from jax.experimental.pallas import tpu as pltpu
from jax.experimental.pallas import tpu_sc as plsc

_N = 1000000
_H = 16
_C = 6
_B = 16384

_NC = 2
_NS = 16
_NW = _NC * _NS
_BPW = _B // _NW
_PH = _BPW // 2

_DB = 8000


def _depad_body(u_ref, o_ref):
  u3 = u_ref[...].reshape(_DB // 8, 8, _H)
  for j in range(8):
    o_ref[:, j * _H:(j + 1) * _H] = u3[:, j, :]


_tc_repack = pl.pallas_call(
    _depad_body,
    grid=(_N // _DB,),
    in_specs=[pl.BlockSpec((_DB, _H), lambda i: (i, 0))],
    out_specs=pl.BlockSpec((_DB // 8, 128), lambda i: (i, 0)),
    out_shape=jax.ShapeDtypeStruct((_N // 8, 128), jnp.float32),
)


def _body(ur8_hbm, urf_hbm, vr8_hbm, vrf_hbm, w_hbm, u_tab, v_tab, out_hbm,
          ur8_a, ur8_b, vr8_a, vr8_b, urf_v, vrf_v, rows_u, rows_v, w_v,
          out_v, sem_u, sem_v):
  wid = lax.axis_index("s") * _NC + lax.axis_index("c")
  base = wid * _BPW
  pltpu.sync_copy(ur8_hbm.at[wid, pl.ds(0, _PH)], ur8_a)
  pltpu.sync_copy(ur8_hbm.at[wid, pl.ds(_PH, _PH)], ur8_b)
  pltpu.sync_copy(vr8_hbm.at[wid, pl.ds(0, _PH)], vr8_a)
  pltpu.sync_copy(vr8_hbm.at[wid, pl.ds(_PH, _PH)], vr8_b)
  pltpu.sync_copy(urf_hbm.at[wid], urf_v)
  pltpu.sync_copy(vrf_hbm.at[wid], vrf_v)
  pltpu.sync_copy(w_hbm, w_v)

  for phase in range(2):
    p0 = phase * _PH
    ur8_v = ur8_a if phase == 0 else ur8_b
    vr8_v = vr8_a if phase == 0 else vr8_b
    cu = pltpu.async_copy(u_tab.at[ur8_v], rows_u, sem_u)
    cv = pltpu.async_copy(v_tab.at[vr8_v], rows_v, sem_v)
    cu.wait()
    cv.wait()

    def blk_body(blk, _, p0=p0):
      j0 = blk * 16
      ru_vec = urf_v[pl.ds(p0 + j0, 16)]
      rv_vec = vrf_v[pl.ds(p0 + j0, 16)]
      for k in range(16):
        offu = (ru_vec[k] & 7) * 16
        offv = (rv_vec[k] & 7) * 16
        u_row = rows_u[j0 + k, pl.ds(offu, 16)]
        v_row = rows_v[j0 + k, pl.ds(offv, 16)]
        acc = w_v[2 * _H, :]
        for h in range(_H):
          acc = acc + u_row[h] * w_v[h, :]
          acc = acc + v_row[h] * w_v[_H + h, :]
        out_v[j0 + k, :] = acc
      return 0

    lax.fori_loop(0, _PH // 16, blk_body, 0)

    pltpu.sync_copy(out_v, out_hbm.at[pl.ds(base + p0, _PH)])


_sc_fused = pl.kernel(
    _body,
    out_type=jax.ShapeDtypeStruct((_B, _H), jnp.float32),
    mesh=plsc.VectorSubcoreMesh(core_axis_name="c", subcore_axis_name="s"),
    scratch_types=[
        pltpu.VMEM((_PH,), jnp.int32),
        pltpu.VMEM((_PH,), jnp.int32),
        pltpu.VMEM((_PH,), jnp.int32),
        pltpu.VMEM((_PH,), jnp.int32),
        pltpu.VMEM((_BPW,), jnp.int32),
        pltpu.VMEM((_BPW,), jnp.int32),
        pltpu.VMEM((_PH, 128), jnp.float32),
        pltpu.VMEM((_PH, 128), jnp.float32),
        pltpu.VMEM((2 * _H + 1, 16), jnp.float32),
        pltpu.VMEM((_PH, _H), jnp.float32),
        pltpu.SemaphoreType.DMA,
        pltpu.SemaphoreType.DMA,
    ],
)


@jax.jit
def kernel(X_batch, U, V, W, b):
  x0 = X_batch[:, 0].astype(jnp.int32)
  x1 = X_batch[:, 1].astype(jnp.int32)
  ur8 = (x0 >> 3).reshape(_NW, _BPW)
  vr8 = (x1 >> 3).reshape(_NW, _BPW)
  urf = x0.reshape(_NW, _BPW)
  vrf = x1.reshape(_NW, _BPW)
  wpad = jnp.zeros((2 * _H + 1, 16), jnp.float32)
  wpad = wpad.at[:2 * _H, :_C].set(W)
  wpad = wpad.at[2 * _H, :_C].set(b)
  u2 = _tc_repack(U)
  v2 = _tc_repack(V)
  out16 = _sc_fused(ur8, urf, vr8, vrf, wpad, u2, v2)
  return out16[:, :_C]


# speedup vs baseline: 1.9536x; 1.9536x over previous
import jax
import jax.numpy as jnp
from jax import lax
from jax.experimental import pallas as pl
from jax.experimental.pallas import tpu as pltpu
from jax.experimental.pallas import tpu_sc as plsc

_N = 1000000
_H = 16
_C = 6
_B = 16384

_NC = 2
_NS = 16
_NW = _NC * _NS
_BPW = _B // _NW
_NBLK = _BPW // 16


def _body(r_hbm, w_hbm, tab, out_hbm, r_v, rows, w_v, out_t, sem):
  wid = lax.axis_index("s") * _NC + lax.axis_index("c")
  base = wid * _BPW
  pltpu.sync_copy(r_hbm.at[wid], r_v)
  pltpu.sync_copy(w_hbm, w_v)

  iota = lax.broadcasted_iota(jnp.int32, (16,), 0)

  def issue_block(blk):
    j0 = blk * 16
    r_vec = r_v[pl.ds(j0, 16)]
    for k in range(16):
      r_k = jnp.sum(jnp.where(iota == k, r_vec, 0))
      pltpu.async_copy(tab.at[r_k], rows.at[j0 + k], sem)

  def drain_block():
    for _ in range(16):
      pltpu.make_async_copy(tab.at[0], rows.at[0], sem).wait()

  def compute_block(blk):
    j0 = blk * 16
    rowv = j0 + iota
    acc = [w_v[_H * _C + c, :] for c in range(_C)]
    for h in range(_H):
      hv = jnp.full((16,), h, dtype=jnp.int32)
      c16 = plsc.load_gather(rows, [rowv, hv])
      for c in range(_C):
        acc[c] = acc[c] + c16 * w_v[h * _C + c, :]
    for c in range(_C):
      out_t[c, pl.ds(j0, 16)] = acc[c]

  issue_block(0)

  def body(blk, _):
    issue_block(blk + 1)
    drain_block()
    compute_block(blk)
    return 0

  lax.fori_loop(0, _NBLK - 1, body, 0)
  drain_block()
  compute_block(_NBLK - 1)

  pltpu.sync_copy(out_t, out_hbm.at[:, pl.ds(base, _BPW)])


_sc_partial = pl.kernel(
    _body,
    out_type=jax.ShapeDtypeStruct((_C, _B), jnp.float32),
    mesh=plsc.VectorSubcoreMesh(core_axis_name="c", subcore_axis_name="s"),
    compiler_params=pltpu.CompilerParams(needs_layout_passes=False),
    scratch_types=[
        pltpu.VMEM((_BPW,), jnp.int32),
        pltpu.VMEM((_BPW, _H), jnp.float32),
        pltpu.VMEM((_H * _C + _C, 16), jnp.float32),
        pltpu.VMEM((_C, _BPW), jnp.float32),
        pltpu.SemaphoreType.DMA,
    ],
)


@jax.jit
def kernel(X_batch, U, V, W, b):
  x0 = X_batch[:, 0].astype(jnp.int32)
  x1 = X_batch[:, 1].astype(jnp.int32)
  ur = x0.reshape(_NW, _BPW)
  vr = x1.reshape(_NW, _BPW)
  wu = jnp.broadcast_to(
      jnp.concatenate([W[:_H].reshape(-1), jnp.zeros((_C,), jnp.float32)]
                      )[:, None], (_H * _C + _C, 16))
  wv = jnp.broadcast_to(
      jnp.concatenate([W[_H:].reshape(-1), b])[:, None], (_H * _C + _C, 16))
  part_u = _sc_partial(ur, wu, U)
  part_v = _sc_partial(vr, wv, V)
  return (part_u + part_v).T
